# baseline (speedup 1.0000x reference)
import jax
import jax.numpy as jnp
from jax.experimental import pallas as pl
from jax.experimental.pallas import tpu as pltpu

_ROWS = 393600
_D = 64


def _copy_body(x_ref, o_ref, sem):
    xf = x_ref.reshape(_ROWS, _D)
    of = o_ref.reshape(_ROWS, _D)
    copy = pltpu.make_async_copy(xf, of, sem)
    copy.start()
    copy.wait()


def kernel(x):
    return pl.pallas_call(
        _copy_body,
        in_specs=[pl.BlockSpec(memory_space=pl.ANY)],
        out_specs=pl.BlockSpec(memory_space=pl.ANY),
        out_shape=jax.ShapeDtypeStruct(x.shape, x.dtype),
        scratch_shapes=[pltpu.SemaphoreType.DMA],
    )(x)


# speedup vs baseline: 14.5846x; 14.5846x over previous
import jax
import jax.numpy as jnp
from jax.experimental import pallas as pl
from jax.experimental.pallas import tpu as pltpu

_ROWS = 393600
_D = 64
_NCHUNK = 40
_CH = _ROWS // _NCHUNK
_K = 4


def _copy_body(x_ref, o_ref, buf, in_sems, out_sems):
    xf = x_ref.reshape(_ROWS, _D)
    of = o_ref.reshape(_ROWS, _D)

    def src(i):
        return xf.at[pl.ds(i * _CH, _CH)]

    def dst(i):
        return of.at[pl.ds(i * _CH, _CH)]

    for k in range(_K):
        pltpu.make_async_copy(src(k), buf.at[k], in_sems.at[k]).start()

    for g in range(_NCHUNK // _K):
        base = g * _K
        for k in range(_K):
            i = base + k
            pltpu.make_async_copy(src(i), buf.at[k], in_sems.at[k]).wait()
            pltpu.make_async_copy(buf.at[k], dst(i), out_sems.at[k]).start()
        for k in range(_K):
            i = base + k
            pltpu.make_async_copy(buf.at[k], dst(i), out_sems.at[k]).wait()
            if i + _K < _NCHUNK:
                pltpu.make_async_copy(
                    src(i + _K), buf.at[k], in_sems.at[k]
                ).start()


def kernel(x):
    return pl.pallas_call(
        _copy_body,
        in_specs=[pl.BlockSpec(memory_space=pl.ANY)],
        out_specs=pl.BlockSpec(memory_space=pl.ANY),
        out_shape=jax.ShapeDtypeStruct(x.shape, x.dtype),
        scratch_shapes=[
            pltpu.VMEM((_K, _CH, _D), jnp.float32),
            pltpu.SemaphoreType.DMA((_K,)),
            pltpu.SemaphoreType.DMA((_K,)),
        ],
    )(x)
